# baseline (speedup 1.0000x reference)
import jax
import jax.numpy as jnp
from jax import lax
from jax.experimental import pallas as pl
from jax.experimental.pallas import tpu as pltpu
from jax.experimental.pallas import tpu_sc as plsc

N = 10000
NP = 10240
E = 320000
FQ = 64
NQ = 4
NC = 2
NS = 16
EPT = E // NS
B = 128
NBUF = 4
NB = -(-EPT // (B * NBUF)) * NBUF
EPT_PAD = NB * B
PAD_ROW = N
RPT = NP // NS


def _scatter_kernel(h_hbm, src_hbm, dst_hbm, out_hbm, src_v, dst_v,
                    r0, r1, r2, r3, zb, acc_sh,
                    g0, g1, g2, g3, s0, s1, s2, s3):
    cidx = lax.axis_index("c")
    sidx = lax.axis_index("s")
    rows = [r0, r1, r2, r3]
    gsem = [g0, g1, g2, g3]
    ssem = [s0, s1, s2, s3]

    pltpu.sync_copy(dst_hbm.at[sidx], dst_v)

    zeros32 = jnp.zeros((32,), jnp.bfloat16)

    def zero_body(i, _):
        for j in range(FQ // 32):
            zb[i, pl.ds(j * 32, 32)] = zeros32
        return 0

    lax.fori_loop(0, B, zero_body, 0)

    for p in range(2):
        q = 2 * cidx + p
        pltpu.sync_copy(src_hbm.at[q, sidx], src_v)
        for k in range(RPT // B):
            pltpu.sync_copy(zb, acc_sh.at[pl.ds(sidx * RPT + k * B, B)])
        plsc.subcore_barrier()

        for b in range(NBUF):
            pltpu.async_copy(h_hbm.at[src_v.at[b]], rows[b], gsem[b])

        def body(g, _):
            for b in range(NBUF):
                j = g * NBUF + b
                pltpu.make_async_copy(h_hbm.at[src_v.at[j]], rows[b],
                                      gsem[b]).wait()
                pltpu.async_copy(rows[b], acc_sh.at[dst_v.at[j]], ssem[b],
                                 add=True)
                pltpu.make_async_copy(rows[b], acc_sh.at[dst_v.at[j]],
                                      ssem[b]).wait()
                pltpu.async_copy(h_hbm.at[src_v.at[j + NBUF]], rows[b],
                                 gsem[b])
            return 0

        lax.fori_loop(0, NB // NBUF - 1, body, 0)
        for b in range(NBUF):
            j = NB - NBUF + b
            pltpu.make_async_copy(h_hbm.at[src_v.at[j]], rows[b],
                                  gsem[b]).wait()
            pltpu.async_copy(rows[b], acc_sh.at[dst_v.at[j]], ssem[b],
                             add=True)
            pltpu.make_async_copy(rows[b], acc_sh.at[dst_v.at[j]],
                                  ssem[b]).wait()
        plsc.subcore_barrier()

        pltpu.sync_copy(acc_sh.at[pl.ds(sidx * RPT, RPT)],
                        out_hbm.at[q, pl.ds(sidx * RPT, RPT)])


_scatter = pl.kernel(
    _scatter_kernel,
    out_type=jax.ShapeDtypeStruct((NQ, NP, FQ), jnp.bfloat16),
    mesh=plsc.VectorSubcoreMesh(core_axis_name="c", subcore_axis_name="s"),
    scratch_types=(
        [pltpu.VMEM((NB, B), jnp.int32),
         pltpu.VMEM((NB, B), jnp.int32)]
        + [pltpu.VMEM((B, FQ), jnp.bfloat16) for _ in range(NBUF)]
        + [pltpu.VMEM((B, FQ), jnp.bfloat16),
           pltpu.VMEM_SHARED((NP, FQ), jnp.bfloat16)]
        + [pltpu.SemaphoreType.DMA for _ in range(2 * NBUF)]
    ),
    compiler_params=pltpu.CompilerParams(use_tc_tiling_on_sc=False),
)


def _mm1_body(x_ref, w_ref, out_ref):
    out_ref[0] = jnp.dot(x_ref[...], w_ref[0],
                         preferred_element_type=jnp.float32).astype(jnp.bfloat16)


def _mm_body(acc_ref, b_ref, w_ref, out_ref):
    w = w_ref[0]
    out = jnp.zeros(out_ref.shape[1:], jnp.float32)
    for q in range(NQ):
        g = jnp.maximum(acc_ref[q].astype(jnp.float32) + b_ref[q], 0.0)
        out = out + jnp.dot(g, w[q * FQ:(q + 1) * FQ],
                            preferred_element_type=jnp.float32)
    out_ref[0] = out.astype(jnp.bfloat16)


def _final_body(acc_ref, b_ref, out_ref):
    out_ref[...] = jnp.concatenate(
        [jnp.maximum(acc_ref[q].astype(jnp.float32) + b_ref[q], 0.0)
         for q in range(NQ)], axis=1)


BN = 1024
NBLK = NP // BN


def _mm1(x_pad, W1q):
    return pl.pallas_call(
        _mm1_body,
        grid=(NQ, NBLK),
        in_specs=[
            pl.BlockSpec((BN, 128), lambda q, i: (i, 0)),
            pl.BlockSpec((1, 128, FQ), lambda q, i: (q, 0, 0)),
        ],
        out_specs=pl.BlockSpec((1, BN, FQ), lambda q, i: (q, i, 0)),
        out_shape=jax.ShapeDtypeStruct((NQ, NP, FQ), jnp.bfloat16),
    )(x_pad, W1q)


def _mm(acc, b_prev, Wq):
    return pl.pallas_call(
        _mm_body,
        grid=(NQ, NBLK),
        in_specs=[
            pl.BlockSpec((NQ, BN, FQ), lambda q, i: (0, i, 0)),
            pl.BlockSpec((NQ, FQ), lambda q, i: (0, 0)),
            pl.BlockSpec((1, NQ * FQ, FQ), lambda q, i: (q, 0, 0)),
        ],
        out_specs=pl.BlockSpec((1, BN, FQ), lambda q, i: (q, i, 0)),
        out_shape=jax.ShapeDtypeStruct((NQ, NP, FQ), jnp.bfloat16),
    )(acc, b_prev, Wq)


FBN = 1000
FNBLK = N // FBN


def _final(acc, b_last):
    return pl.pallas_call(
        _final_body,
        grid=(FNBLK,),
        in_specs=[
            pl.BlockSpec((NQ, FBN, FQ), lambda i: (0, i, 0)),
            pl.BlockSpec((NQ, FQ), lambda i: (0, 0)),
        ],
        out_specs=pl.BlockSpec((FBN, NQ * FQ), lambda i: (i, 0)),
        out_shape=jax.ShapeDtypeStruct((N, NQ * FQ), jnp.float32),
    )(acc, b_last)


@jax.jit
def kernel(x, edge_index, W1, b1, W2, b2, W3, b3):
    src = edge_index[0]
    dst = edge_index[1]

    pad = EPT_PAD - EPT
    src_sl = jnp.pad(src.reshape(NS, EPT), ((0, 0), (0, pad)))
    dst_sl = jnp.pad(dst.reshape(NS, EPT), ((0, 0), (0, pad)),
                     constant_values=PAD_ROW)
    src4d = jnp.stack([src_sl + q * NP for q in range(NQ)]).reshape(NQ, NS, NB, B)
    dst3d = dst_sl.reshape(NS, NB, B)

    x_pad = jnp.pad(x, ((0, NP - N), (0, 0)))
    b1h = b1.reshape(NQ, FQ)
    b2h = b2.reshape(NQ, FQ)
    b3h = b3.reshape(NQ, FQ)
    W1q = W1.reshape(128, NQ, FQ).transpose(1, 0, 2)
    W2q = W2.reshape(256, NQ, FQ).transpose(1, 0, 2)
    W3q = W3.reshape(256, NQ, FQ).transpose(1, 0, 2)

    h = _mm1(x_pad, W1q)
    acc = _scatter(h.reshape(NQ * NP, FQ), src4d, dst3d)
    h = _mm(acc, b1h, W2q)
    acc = _scatter(h.reshape(NQ * NP, FQ), src4d, dst3d)
    h = _mm(acc, b2h, W3q)
    acc = _scatter(h.reshape(NQ * NP, FQ), src4d, dst3d)
    return _final(acc, b3h)


# speedup vs baseline: 1.1559x; 1.1559x over previous
import jax
import jax.numpy as jnp
from jax import lax
from jax.experimental import pallas as pl
from jax.experimental.pallas import tpu as pltpu
from jax.experimental.pallas import tpu_sc as plsc

N = 10000
NP = 10240
E = 320000
FH = 128
NH = 2
NS = 16
EPT = E // NS
B = 128
NBUF = 4
NB = -(-EPT // (B * NBUF)) * NBUF
EPT_PAD = NB * B
PAD_ROW = N
RPT = NP // NS


def _scatter_kernel(h_hbm, src_hbm, dst_hbm, out_hbm, src_v, dst_v,
                    r0, r1, r2, r3, zb, acc_sh,
                    g0, g1, g2, g3, s0, s1, s2, s3):
    cidx = lax.axis_index("c")
    sidx = lax.axis_index("s")
    rows = [r0, r1, r2, r3]
    gsem = [g0, g1, g2, g3]
    ssem = [s0, s1, s2, s3]

    pltpu.sync_copy(src_hbm.at[cidx, sidx], src_v)
    pltpu.sync_copy(dst_hbm.at[sidx], dst_v)

    zeros32 = jnp.zeros((32,), jnp.bfloat16)

    def zero_body(i, _):
        for j in range(FH // 32):
            zb[i, pl.ds(j * 32, 32)] = zeros32
        return 0

    lax.fori_loop(0, B, zero_body, 0)
    for k in range(RPT // B):
        pltpu.sync_copy(zb, acc_sh.at[pl.ds(sidx * RPT + k * B, B)])
    plsc.subcore_barrier()

    for b in range(NBUF):
        pltpu.async_copy(h_hbm.at[src_v.at[b]], rows[b], gsem[b])

    def body(g, _):
        for b in range(NBUF):
            j = g * NBUF + b
            pltpu.make_async_copy(h_hbm.at[src_v.at[j]], rows[b],
                                  gsem[b]).wait()
            pltpu.async_copy(rows[b], acc_sh.at[dst_v.at[j]], ssem[b],
                             add=True)
            pltpu.make_async_copy(rows[b], acc_sh.at[dst_v.at[j]],
                                  ssem[b]).wait()
            pltpu.async_copy(h_hbm.at[src_v.at[j + NBUF]], rows[b],
                             gsem[b])
        return 0

    lax.fori_loop(0, NB // NBUF - 1, body, 0)
    for b in range(NBUF):
        j = NB - NBUF + b
        pltpu.make_async_copy(h_hbm.at[src_v.at[j]], rows[b],
                              gsem[b]).wait()
        pltpu.async_copy(rows[b], acc_sh.at[dst_v.at[j]], ssem[b],
                         add=True)
        pltpu.make_async_copy(rows[b], acc_sh.at[dst_v.at[j]],
                              ssem[b]).wait()
    plsc.subcore_barrier()

    pltpu.sync_copy(acc_sh.at[pl.ds(sidx * RPT, RPT)],
                    out_hbm.at[cidx, pl.ds(sidx * RPT, RPT)])


_scatter = pl.kernel(
    _scatter_kernel,
    out_type=jax.ShapeDtypeStruct((NH, NP, FH), jnp.bfloat16),
    mesh=plsc.VectorSubcoreMesh(core_axis_name="c", subcore_axis_name="s"),
    scratch_types=(
        [pltpu.VMEM((NB, B), jnp.int32),
         pltpu.VMEM((NB, B), jnp.int32)]
        + [pltpu.VMEM((B, FH), jnp.bfloat16) for _ in range(NBUF)]
        + [pltpu.VMEM((B, FH), jnp.bfloat16),
           pltpu.VMEM_SHARED((NP, FH), jnp.bfloat16)]
        + [pltpu.SemaphoreType.DMA for _ in range(2 * NBUF)]
    ),
    compiler_params=pltpu.CompilerParams(use_tc_tiling_on_sc=False),
)


def _mm1_body(x_ref, w_ref, out_ref):
    out_ref[0] = jnp.dot(x_ref[...], w_ref[0],
                         preferred_element_type=jnp.float32).astype(jnp.bfloat16)


def _mm_body(acc_ref, b_ref, w_ref, out_ref):
    w = w_ref[0]
    out = jnp.zeros(out_ref.shape[1:], jnp.float32)
    for c in range(NH):
        g = jnp.maximum(acc_ref[c].astype(jnp.float32) + b_ref[c], 0.0)
        out = out + jnp.dot(g, w[c * FH:(c + 1) * FH],
                            preferred_element_type=jnp.float32)
    out_ref[0] = out.astype(jnp.bfloat16)


def _final_body(acc_ref, b_ref, out_ref):
    out_ref[...] = jnp.concatenate(
        [jnp.maximum(acc_ref[c].astype(jnp.float32) + b_ref[c], 0.0)
         for c in range(NH)], axis=1)


BN = 1024
NBLK = NP // BN


def _mm1(x_pad, W1h):
    return pl.pallas_call(
        _mm1_body,
        grid=(NH, NBLK),
        in_specs=[
            pl.BlockSpec((BN, 128), lambda c, i: (i, 0)),
            pl.BlockSpec((1, 128, FH), lambda c, i: (c, 0, 0)),
        ],
        out_specs=pl.BlockSpec((1, BN, FH), lambda c, i: (c, i, 0)),
        out_shape=jax.ShapeDtypeStruct((NH, NP, FH), jnp.bfloat16),
    )(x_pad, W1h)


def _mm(acc, b_prev, Wh):
    return pl.pallas_call(
        _mm_body,
        grid=(NH, NBLK),
        in_specs=[
            pl.BlockSpec((NH, BN, FH), lambda c, i: (0, i, 0)),
            pl.BlockSpec((NH, FH), lambda c, i: (0, 0)),
            pl.BlockSpec((1, NH * FH, FH), lambda c, i: (c, 0, 0)),
        ],
        out_specs=pl.BlockSpec((1, BN, FH), lambda c, i: (c, i, 0)),
        out_shape=jax.ShapeDtypeStruct((NH, NP, FH), jnp.bfloat16),
    )(acc, b_prev, Wh)


FBN = 1000
FNBLK = N // FBN


def _final(acc, b_last):
    return pl.pallas_call(
        _final_body,
        grid=(FNBLK,),
        in_specs=[
            pl.BlockSpec((NH, FBN, FH), lambda i: (0, i, 0)),
            pl.BlockSpec((NH, FH), lambda i: (0, 0)),
        ],
        out_specs=pl.BlockSpec((FBN, NH * FH), lambda i: (i, 0)),
        out_shape=jax.ShapeDtypeStruct((N, NH * FH), jnp.float32),
    )(acc, b_last)


@jax.jit
def kernel(x, edge_index, W1, b1, W2, b2, W3, b3):
    src = edge_index[0]
    dst = edge_index[1]

    pad = EPT_PAD - EPT
    src_sl = jnp.pad(src.reshape(NS, EPT), ((0, 0), (0, pad)))
    dst_sl = jnp.pad(dst.reshape(NS, EPT), ((0, 0), (0, pad)),
                     constant_values=PAD_ROW)
    src4d = jnp.stack([src_sl + c * NP for c in range(NH)]).reshape(NH, NS, NB, B)
    dst3d = dst_sl.reshape(NS, NB, B)

    x_pad = jnp.pad(x, ((0, NP - N), (0, 0)))
    b1h = b1.reshape(NH, FH)
    b2h = b2.reshape(NH, FH)
    b3h = b3.reshape(NH, FH)
    W1h = W1.reshape(128, NH, FH).transpose(1, 0, 2)
    W2h = W2.reshape(256, NH, FH).transpose(1, 0, 2)
    W3h = W3.reshape(256, NH, FH).transpose(1, 0, 2)

    h = _mm1(x_pad, W1h)
    acc = _scatter(h.reshape(NH * NP, FH), src4d, dst3d)
    h = _mm(acc, b1h, W2h)
    acc = _scatter(h.reshape(NH * NP, FH), src4d, dst3d)
    h = _mm(acc, b2h, W3h)
    acc = _scatter(h.reshape(NH * NP, FH), src4d, dst3d)
    return _final(acc, b3h)


# speedup vs baseline: 1.1626x; 1.0058x over previous
import jax
import jax.numpy as jnp
from jax import lax
from jax.experimental import pallas as pl
from jax.experimental.pallas import tpu as pltpu
from jax.experimental.pallas import tpu_sc as plsc

N = 10000
NP = 10240
E = 320000
FH = 128
NH = 2
NS = 16
EPT = E // NS
B = 128
NBUF = 4
NB = -(-EPT // (B * NBUF)) * NBUF
EPT_PAD = NB * B
PAD_ROW = N
RPT = NP // NS


def _scatter_kernel(h_hbm, src_hbm, dst_hbm, out_hbm, src_v, dst_v, *rest):
    cidx = lax.axis_index("c")
    sidx = lax.axis_index("s")
    rows = list(rest[:NBUF])
    zb = rest[NBUF]
    acc_sh = rest[NBUF + 1]
    gsem = list(rest[NBUF + 2:NBUF + 2 + NBUF])
    ssem = list(rest[NBUF + 2 + NBUF:])

    pltpu.sync_copy(src_hbm.at[sidx], src_v)
    pltpu.sync_copy(dst_hbm.at[sidx], dst_v)

    off = jnp.full((16,), NP, jnp.int32) * cidx

    def off_body(i, _):
        for j in range(B // 16):
            sl = pl.ds(j * 16, 16)
            src_v[i, sl] = src_v[i, sl] + off
        return 0

    lax.fori_loop(0, NB, off_body, 0)

    zeros32 = jnp.zeros((32,), jnp.bfloat16)

    def zero_body(i, _):
        for j in range(FH // 32):
            zb[i, pl.ds(j * 32, 32)] = zeros32
        return 0

    lax.fori_loop(0, B, zero_body, 0)
    for k in range(RPT // B):
        pltpu.sync_copy(zb, acc_sh.at[pl.ds(sidx * RPT + k * B, B)])
    plsc.subcore_barrier()

    for b in range(NBUF):
        pltpu.async_copy(h_hbm.at[src_v.at[b]], rows[b], gsem[b])

    def body(g, _):
        for b in range(NBUF):
            j = g * NBUF + b
            pltpu.make_async_copy(h_hbm.at[src_v.at[j]], rows[b],
                                  gsem[b]).wait()
            pltpu.async_copy(rows[b], acc_sh.at[dst_v.at[j]], ssem[b],
                             add=True)
            pltpu.make_async_copy(rows[b], acc_sh.at[dst_v.at[j]],
                                  ssem[b]).wait()
            pltpu.async_copy(h_hbm.at[src_v.at[j + NBUF]], rows[b],
                             gsem[b])
        return 0

    lax.fori_loop(0, NB // NBUF - 1, body, 0)
    for b in range(NBUF):
        j = NB - NBUF + b
        pltpu.make_async_copy(h_hbm.at[src_v.at[j]], rows[b],
                              gsem[b]).wait()
        pltpu.async_copy(rows[b], acc_sh.at[dst_v.at[j]], ssem[b],
                         add=True)
        pltpu.make_async_copy(rows[b], acc_sh.at[dst_v.at[j]],
                              ssem[b]).wait()
    plsc.subcore_barrier()

    pltpu.sync_copy(acc_sh.at[pl.ds(sidx * RPT, RPT)],
                    out_hbm.at[cidx, pl.ds(sidx * RPT, RPT)])


_scatter = pl.kernel(
    _scatter_kernel,
    out_type=jax.ShapeDtypeStruct((NH, NP, FH), jnp.bfloat16),
    mesh=plsc.VectorSubcoreMesh(core_axis_name="c", subcore_axis_name="s"),
    scratch_types=(
        [pltpu.VMEM((NB, B), jnp.int32),
         pltpu.VMEM((NB, B), jnp.int32)]
        + [pltpu.VMEM((B, FH), jnp.bfloat16) for _ in range(NBUF)]
        + [pltpu.VMEM((B, FH), jnp.bfloat16),
           pltpu.VMEM_SHARED((NP, FH), jnp.bfloat16)]
        + [pltpu.SemaphoreType.DMA for _ in range(2 * NBUF)]
    ),
    compiler_params=pltpu.CompilerParams(use_tc_tiling_on_sc=False),
)


def _mm1_body(x_ref, w_ref, out_ref):
    out_ref[0] = jnp.dot(x_ref[...], w_ref[...],
                         preferred_element_type=jnp.float32).astype(jnp.bfloat16)


def _mm_body(acc_ref, b_ref, w_ref, out_ref):
    g = jnp.concatenate(
        [jnp.maximum(acc_ref[c].astype(jnp.float32) + b_ref[c], 0.0)
         for c in range(NH)], axis=1)
    out_ref[0] = jnp.dot(g, w_ref[...],
                         preferred_element_type=jnp.float32).astype(jnp.bfloat16)


def _final_body(acc_ref, b_ref, out_ref):
    out_ref[...] = jnp.concatenate(
        [jnp.maximum(acc_ref[c].astype(jnp.float32) + b_ref[c], 0.0)
         for c in range(NH)], axis=1)


BN = 1024
NBLK = NP // BN


def _mm1(x_pad, W1h):
    return pl.pallas_call(
        _mm1_body,
        grid=(NH, NBLK),
        in_specs=[
            pl.BlockSpec((BN, 128), lambda c, i: (i, 0)),
            pl.BlockSpec((128, FH), lambda c, i: (0, c)),
        ],
        out_specs=pl.BlockSpec((1, BN, FH), lambda c, i: (c, i, 0)),
        out_shape=jax.ShapeDtypeStruct((NH, NP, FH), jnp.bfloat16),
    )(x_pad, W1h)


def _mm(acc, b_prev, Wh):
    return pl.pallas_call(
        _mm_body,
        grid=(NH, NBLK),
        in_specs=[
            pl.BlockSpec((NH, BN, FH), lambda c, i: (0, i, 0)),
            pl.BlockSpec((NH, FH), lambda c, i: (0, 0)),
            pl.BlockSpec((NH * FH, FH), lambda c, i: (0, c)),
        ],
        out_specs=pl.BlockSpec((1, BN, FH), lambda c, i: (c, i, 0)),
        out_shape=jax.ShapeDtypeStruct((NH, NP, FH), jnp.bfloat16),
    )(acc, b_prev, Wh)


FBN = 1000
FNBLK = N // FBN


def _final(acc, b_last):
    return pl.pallas_call(
        _final_body,
        grid=(FNBLK,),
        in_specs=[
            pl.BlockSpec((NH, FBN, FH), lambda i: (0, i, 0)),
            pl.BlockSpec((NH, FH), lambda i: (0, 0)),
        ],
        out_specs=pl.BlockSpec((FBN, NH * FH), lambda i: (i, 0)),
        out_shape=jax.ShapeDtypeStruct((N, NH * FH), jnp.float32),
    )(acc, b_last)


@jax.jit
def kernel(x, edge_index, W1, b1, W2, b2, W3, b3):
    src = edge_index[0]
    dst = edge_index[1]

    pad = EPT_PAD - EPT
    src_sl = jnp.pad(src.reshape(NS, EPT), ((0, 0), (0, pad)))
    dst_sl = jnp.pad(dst.reshape(NS, EPT), ((0, 0), (0, pad)),
                     constant_values=PAD_ROW)
    src3d = src_sl.reshape(NS, NB, B)
    dst3d = dst_sl.reshape(NS, NB, B)

    x_pad = jnp.pad(x, ((0, NP - N), (0, 0)))
    b1h = b1.reshape(NH, FH)
    b2h = b2.reshape(NH, FH)
    b3h = b3.reshape(NH, FH)
    h = _mm1(x_pad, W1)
    acc = _scatter(h.reshape(NH * NP, FH), src3d, dst3d)
    h = _mm(acc, b1h, W2)
    acc = _scatter(h.reshape(NH * NP, FH), src3d, dst3d)
    h = _mm(acc, b2h, W3)
    acc = _scatter(h.reshape(NH * NP, FH), src3d, dst3d)
    return _final(acc, b3h)
